# baseline (speedup 1.0000x reference)
import functools

import jax
import jax.numpy as jnp
from jax import lax
from jax.experimental import pallas as pl
from jax.experimental.pallas import tpu as pltpu
from jax.experimental.pallas import tpu_sc as plsc

B = 128
P = 10
K = 5
LP = 5
D = 1000
PE_W = (K * LP + 1) * D
PR_W = K * LP * D
KR_W = K * D
NW = 32
RPW = B // NW
A_W = 24960
A_CH = 8320
SLAB_W = PE_W - A_W


def _l2n(v):
    return v * lax.rsqrt(jnp.maximum(jnp.sum(v * v, axis=1, keepdims=True), 1e-12))


def _tc_body(x_ref, pf_ref, pk_ref,
             idx_ref, pn_ref, xn_ref, sim_ref, rs_ref,
             pr8a_ref, slab_ref, bk8_ref):
    x = x_ref[...]
    pk = pk_ref[...]
    pf = pf_ref[...]

    pn = _l2n(pk)
    xn = _l2n(x)
    sim = lax.dot_general(xn, pn, (((1,), (1,)), ((), ())))

    colj = lax.broadcasted_iota(jnp.int32, (1, P), 1)
    rank = jnp.zeros((B, P), jnp.int32)
    for jp in range(P):
        sj = sim[:, jp:jp + 1]
        gt = (sj > sim).astype(jnp.int32)
        eq = (sj == sim).astype(jnp.int32) * (colj > jp).astype(jnp.int32)
        rank = rank + gt + eq
    in_top = (rank < K).astype(jnp.int32)
    counts = jnp.sum(in_top, axis=0, keepdims=True)

    crank = jnp.zeros((1, P), jnp.int32)
    for jp in range(P):
        cj = counts[:, jp:jp + 1]
        gt = (cj > counts).astype(jnp.int32)
        eq = (cj == counts).astype(jnp.int32) * (colj > jp).astype(jnp.int32)
        crank = crank + gt + eq
    sel = crank < K
    self32 = sel.astype(jnp.float32)

    r_io = lax.broadcasted_iota(jnp.int32, (P, P), 0)
    c_io = lax.broadcasted_iota(jnp.int32, (P, P), 1)
    strict_lt = (r_io < c_io).astype(jnp.float32)
    pos = lax.dot_general(self32, strict_lt, (((1,), (0,)), ((), ())),
                          precision=jax.lax.Precision.HIGHEST)

    s_io = lax.broadcasted_iota(jnp.int32, (K, P), 0).astype(jnp.float32)
    oh = ((s_io == pos) & sel).astype(jnp.float32)

    hi = jax.lax.Precision.HIGHEST
    coljf = colj.astype(jnp.float32)
    major_f = lax.dot_general(coljf, oh, (((1,), (1,)), ((), ())),
                              precision=hi)

    sel_key = lax.dot_general(oh, pn, (((1,), (0,)), ((), ())),
                              precision=hi)
    sel_pr = lax.dot_general(oh, pf, (((1,), (0,)), ((), ())),
                             precision=hi)

    krow = jnp.concatenate([sel_key[s:s + 1, :] for s in range(K)], axis=1)
    prow = jnp.concatenate([sel_pr[s:s + 1, :] for s in range(K)], axis=1)

    pr8a_ref[...] = jnp.broadcast_to(prow[:, :A_W], (8, A_W))
    slab_ref[...] = jnp.concatenate(
        [jnp.broadcast_to(prow[:, A_W:], (B, PR_W - A_W)), x], axis=1)
    bk8_ref[...] = jnp.broadcast_to(krow, (8, KR_W))

    idx_ref[...] = jnp.broadcast_to(major_f.astype(jnp.int32), (B, K))
    pn_ref[...] = pn
    xn_ref[...] = xn
    sim_ref[...] = sim

    ksum = jnp.sum(sel_key, axis=0, keepdims=True)
    xnsum = jnp.sum(xn, axis=0, keepdims=True)
    rs_ref[...] = (jnp.sum(ksum * xnsum) / B).reshape(1, 1)


_mesh = plsc.VectorSubcoreMesh(core_axis_name="c", subcore_axis_name="s")


@functools.partial(
    pl.kernel,
    mesh=_mesh,
    out_type=[
        jax.ShapeDtypeStruct((B, PE_W), jnp.float32),
        jax.ShapeDtypeStruct((B, KR_W), jnp.float32),
    ],
    scratch_types=[
        pltpu.VMEM((8, A_CH), jnp.float32),
        pltpu.VMEM((16, SLAB_W), jnp.float32),
        pltpu.VMEM((8, KR_W), jnp.float32),
        pltpu.SemaphoreType.DMA,
    ],
)
def _sc_broadcast(pr8a_hbm, slab_hbm, bk8_hbm, pe_hbm, bkn_hbm,
                  bufa, bufb, bufc, sem):
    cid = lax.axis_index("c")
    sid = lax.axis_index("s")
    wid = sid * 2 + cid

    @pl.when(wid < 24)
    def _():
        k = wid // 8
        p = wid % 8
        c0 = pl.multiple_of(k * A_CH, 128)
        pltpu.async_copy(pr8a_hbm.at[:, pl.ds(c0, A_CH)], bufa, sem).wait()
        cps = []
        for t in range(2):
            r0 = pl.multiple_of((2 * p + t) * 8, 8)
            cps.append(pltpu.async_copy(
                bufa, pe_hbm.at[pl.ds(r0, 8), pl.ds(c0, A_CH)], sem))
        for cp in cps:
            cp.wait()

    @pl.when(wid >= 24)
    def _():
        g = wid - 24
        r0 = pl.multiple_of(g * 16, 8)
        s0 = pltpu.async_copy(slab_hbm.at[pl.ds(r0, 16), :], bufb, sem)
        s1 = pltpu.async_copy(bk8_hbm, bufc, sem)
        s0.wait()
        s1.wait()
        c0 = pltpu.async_copy(
            bufb, pe_hbm.at[pl.ds(r0, 16), pl.ds(A_W, SLAB_W)], sem)
        r1 = pl.multiple_of(g * 16 + 8, 8)
        c1 = pltpu.async_copy(bufc, bkn_hbm.at[pl.ds(r0, 8), :], sem)
        c2 = pltpu.async_copy(bufc, bkn_hbm.at[pl.ds(r1, 8), :], sem)
        c0.wait()
        c1.wait()
        c2.wait()


def kernel(x, prompt, prompt_key):
    pf = prompt.reshape(P, LP * D)
    idx_b, pn, xn, sim, rs, pr8a, slab, bk8 = pl.pallas_call(
        _tc_body,
        out_shape=[
            jax.ShapeDtypeStruct((B, K), jnp.int32),
            jax.ShapeDtypeStruct((P, D), jnp.float32),
            jax.ShapeDtypeStruct((B, D), jnp.float32),
            jax.ShapeDtypeStruct((B, P), jnp.float32),
            jax.ShapeDtypeStruct((1, 1), jnp.float32),
            jax.ShapeDtypeStruct((8, A_W), jnp.float32),
            jax.ShapeDtypeStruct((B, SLAB_W), jnp.float32),
            jax.ShapeDtypeStruct((8, KR_W), jnp.float32),
        ],
    )(x, pf, prompt_key)

    pe, bkn = _sc_broadcast(pr8a, slab, bk8)
    return (idx_b, pn, xn, sim, bkn.reshape(B, K, D), rs[0, 0], pe)


# speedup vs baseline: 1.0091x; 1.0091x over previous
import functools

import jax
import jax.numpy as jnp
from jax import lax
from jax.experimental import pallas as pl
from jax.experimental.pallas import tpu as pltpu
from jax.experimental.pallas import tpu_sc as plsc

B = 128
P = 10
K = 5
LP = 5
D = 1000
PE_W = (K * LP + 1) * D
PR_W = K * LP * D
KR_W = K * D
NW = 32
RPW = B // NW
A_W = 24960
A_CH = 8320
SLAB_W = PE_W - A_W


def _l2n(v):
    return v * lax.rsqrt(jnp.maximum(jnp.sum(v * v, axis=1, keepdims=True), 1e-12))


def _selection(x, pk, pf):
    pn = _l2n(pk)
    xn = _l2n(x)
    sim = lax.dot_general(xn, pn, (((1,), (1,)), ((), ())))

    colj = lax.broadcasted_iota(jnp.int32, (1, P), 1)
    rank = jnp.zeros((B, P), jnp.int32)
    for jp in range(P):
        sj = sim[:, jp:jp + 1]
        gt = (sj > sim).astype(jnp.int32)
        eq = (sj == sim).astype(jnp.int32) * (colj > jp).astype(jnp.int32)
        rank = rank + gt + eq
    in_top = (rank < K).astype(jnp.int32)
    counts = jnp.sum(in_top, axis=0, keepdims=True)

    crank = jnp.zeros((1, P), jnp.int32)
    for jp in range(P):
        cj = counts[:, jp:jp + 1]
        gt = (cj > counts).astype(jnp.int32)
        eq = (cj == counts).astype(jnp.int32) * (colj > jp).astype(jnp.int32)
        crank = crank + gt + eq
    sel = crank < K
    self32 = sel.astype(jnp.float32)

    hi = jax.lax.Precision.HIGHEST
    r_io = lax.broadcasted_iota(jnp.int32, (P, P), 0)
    c_io = lax.broadcasted_iota(jnp.int32, (P, P), 1)
    strict_lt = (r_io < c_io).astype(jnp.float32)
    pos = lax.dot_general(self32, strict_lt, (((1,), (0,)), ((), ())),
                          precision=hi)

    s_io = lax.broadcasted_iota(jnp.int32, (K, P), 0).astype(jnp.float32)
    oh = ((s_io == pos) & sel).astype(jnp.float32)

    coljf = colj.astype(jnp.float32)
    major_f = lax.dot_general(coljf, oh, (((1,), (1,)), ((), ())),
                              precision=hi)

    sel_key = lax.dot_general(oh, pn, (((1,), (0,)), ((), ())),
                              precision=hi)
    sel_pr = None
    if pf is not None:
        sel_pr = lax.dot_general(oh, pf, (((1,), (0,)), ((), ())),
                                 precision=hi)
    return pn, xn, sim, major_f, sel_key, sel_pr


def _tc_body1(x_ref, pf_ref, pk_ref, pr8a_ref, slab_ref, bk8_ref):
    x = x_ref[...]
    _, _, _, _, sel_key, sel_pr = _selection(x, pk_ref[...], pf_ref[...])
    krow = jnp.concatenate([sel_key[s:s + 1, :] for s in range(K)], axis=1)
    prow = jnp.concatenate([sel_pr[s:s + 1, :] for s in range(K)], axis=1)
    pr8a_ref[...] = jnp.broadcast_to(prow[:, :A_W], (8, A_W))
    slab_ref[...] = jnp.concatenate(
        [jnp.broadcast_to(prow[:, A_W:], (B, PR_W - A_W)), x], axis=1)
    bk8_ref[...] = jnp.broadcast_to(krow, (8, KR_W))


def _tc_body2(x_ref, pk_ref, idx_ref, pn_ref, xn_ref, sim_ref, rs_ref):
    pn, xn, sim, major_f, sel_key, _ = _selection(
        x_ref[...], pk_ref[...], None)
    idx_ref[...] = jnp.broadcast_to(major_f.astype(jnp.int32), (B, K))
    pn_ref[...] = pn
    xn_ref[...] = xn
    sim_ref[...] = sim
    ksum = jnp.sum(sel_key, axis=0, keepdims=True)
    xnsum = jnp.sum(xn, axis=0, keepdims=True)
    rs_ref[...] = (jnp.sum(ksum * xnsum) / B).reshape(1, 1)


_mesh = plsc.VectorSubcoreMesh(core_axis_name="c", subcore_axis_name="s")


@functools.partial(
    pl.kernel,
    mesh=_mesh,
    out_type=[
        jax.ShapeDtypeStruct((B, PE_W), jnp.float32),
        jax.ShapeDtypeStruct((B, KR_W), jnp.float32),
    ],
    scratch_types=[
        pltpu.VMEM((8, A_CH), jnp.float32),
        pltpu.VMEM((16, SLAB_W), jnp.float32),
        pltpu.VMEM((8, KR_W), jnp.float32),
        pltpu.SemaphoreType.DMA,
    ],
)
def _sc_broadcast(pr8a_hbm, slab_hbm, bk8_hbm, pe_hbm, bkn_hbm,
                  bufa, bufb, bufc, sem):
    cid = lax.axis_index("c")
    sid = lax.axis_index("s")
    wid = sid * 2 + cid

    @pl.when(wid < 24)
    def _():
        k = wid // 8
        p = wid % 8
        c0 = pl.multiple_of(k * A_CH, 128)
        pltpu.async_copy(pr8a_hbm.at[:, pl.ds(c0, A_CH)], bufa, sem).wait()
        cps = []
        for t in range(2):
            r0 = pl.multiple_of((2 * p + t) * 8, 8)
            cps.append(pltpu.async_copy(
                bufa, pe_hbm.at[pl.ds(r0, 8), pl.ds(c0, A_CH)], sem))
        for cp in cps:
            cp.wait()

    @pl.when(wid >= 24)
    def _():
        g = wid - 24
        r0 = pl.multiple_of(g * 16, 8)
        s0 = pltpu.async_copy(slab_hbm.at[pl.ds(r0, 16), :], bufb, sem)
        s1 = pltpu.async_copy(bk8_hbm, bufc, sem)
        s0.wait()
        s1.wait()
        c0 = pltpu.async_copy(
            bufb, pe_hbm.at[pl.ds(r0, 16), pl.ds(A_W, SLAB_W)], sem)
        r1 = pl.multiple_of(g * 16 + 8, 8)
        c1 = pltpu.async_copy(bufc, bkn_hbm.at[pl.ds(r0, 8), :], sem)
        c2 = pltpu.async_copy(bufc, bkn_hbm.at[pl.ds(r1, 8), :], sem)
        c0.wait()
        c1.wait()
        c2.wait()


def kernel(x, prompt, prompt_key):
    pf = prompt.reshape(P, LP * D)
    pr8a, slab, bk8 = pl.pallas_call(
        _tc_body1,
        out_shape=[
            jax.ShapeDtypeStruct((8, A_W), jnp.float32),
            jax.ShapeDtypeStruct((B, SLAB_W), jnp.float32),
            jax.ShapeDtypeStruct((8, KR_W), jnp.float32),
        ],
    )(x, pf, prompt_key)

    pe, bkn = _sc_broadcast(pr8a, slab, bk8)

    idx_b, pn, xn, sim, rs = pl.pallas_call(
        _tc_body2,
        out_shape=[
            jax.ShapeDtypeStruct((B, K), jnp.int32),
            jax.ShapeDtypeStruct((P, D), jnp.float32),
            jax.ShapeDtypeStruct((B, D), jnp.float32),
            jax.ShapeDtypeStruct((B, P), jnp.float32),
            jax.ShapeDtypeStruct((1, 1), jnp.float32),
        ],
    )(x, prompt_key)
    return (idx_b, pn, xn, sim, bkn.reshape(B, K, D), rs[0, 0], pe)


# speedup vs baseline: 1.5003x; 1.4867x over previous
import jax
import jax.numpy as jnp
from jax import lax
from jax.experimental import pallas as pl
from jax.experimental.pallas import tpu as pltpu

B = 128
P = 10
K = 5
LP = 5
D = 1000
BLK = 32
GRID = B // BLK
PE_W = (K * LP + 1) * D


def _l2n(v):
    return v * lax.rsqrt(jnp.maximum(jnp.sum(v * v, axis=1, keepdims=True), 1e-12))


def _body(x_ref, pf_ref, pk_ref,
          idx_ref, pn_ref, xn_ref, sim_ref, bkn_ref, rs_ref, pe_ref,
          prow_ref, krow_ref, major_ref, pns_ref):
    i = pl.program_id(0)

    @pl.when(i == 0)
    def _():
        x = x_ref[...]
        pk = pk_ref[...]
        pf = pf_ref[...]

        pn = _l2n(pk)
        xn = _l2n(x)
        sim = lax.dot_general(xn, pn, (((1,), (1,)), ((), ())))

        colj = lax.broadcasted_iota(jnp.int32, (1, P), 1)
        rank = jnp.zeros((B, P), jnp.int32)
        for jp in range(P):
            sj = sim[:, jp:jp + 1]
            gt = (sj > sim).astype(jnp.int32)
            eq = (sj == sim).astype(jnp.int32) * (colj > jp).astype(jnp.int32)
            rank = rank + gt + eq
        in_top = (rank < K).astype(jnp.int32)
        counts = jnp.sum(in_top, axis=0, keepdims=True)

        crank = jnp.zeros((1, P), jnp.int32)
        for jp in range(P):
            cj = counts[:, jp:jp + 1]
            gt = (cj > counts).astype(jnp.int32)
            eq = (cj == counts).astype(jnp.int32) * (colj > jp).astype(jnp.int32)
            crank = crank + gt + eq
        sel = crank < K
        self32 = sel.astype(jnp.float32)

        hi = jax.lax.Precision.HIGHEST
        r_io = lax.broadcasted_iota(jnp.int32, (P, P), 0)
        c_io = lax.broadcasted_iota(jnp.int32, (P, P), 1)
        strict_lt = (r_io < c_io).astype(jnp.float32)
        pos = lax.dot_general(self32, strict_lt, (((1,), (0,)), ((), ())),
                              precision=hi)

        s_io = lax.broadcasted_iota(jnp.int32, (K, P), 0).astype(jnp.float32)
        oh = ((s_io == pos) & sel).astype(jnp.float32)

        coljf = colj.astype(jnp.float32)
        major_f = lax.dot_general(coljf, oh, (((1,), (1,)), ((), ())),
                                  precision=hi)

        sel_key = lax.dot_general(oh, pn, (((1,), (0,)), ((), ())),
                                  precision=hi)
        sel_pr = lax.dot_general(oh, pf, (((1,), (0,)), ((), ())),
                                 precision=hi)

        krow_ref[...] = jnp.concatenate(
            [sel_key[s:s + 1, :] for s in range(K)], axis=1)
        prow_ref[...] = jnp.concatenate(
            [sel_pr[s:s + 1, :] for s in range(K)], axis=1)
        major_ref[...] = major_f.astype(jnp.int32)
        pns_ref[...] = pn
        pn_ref[...] = pn

        ksum = jnp.sum(sel_key, axis=0, keepdims=True)
        xnsum = jnp.sum(xn, axis=0, keepdims=True)
        rs_ref[...] = (jnp.sum(ksum * xnsum) / B).reshape(1, 1)

    x_blk = x_ref[pl.ds(i * BLK, BLK), :]
    xn_blk = _l2n(x_blk)
    pn = pns_ref[...]
    sim_blk = lax.dot_general(xn_blk, pn, (((1,), (1,)), ((), ())))

    idx_ref[...] = jnp.broadcast_to(major_ref[...], (BLK, K))
    xn_ref[...] = xn_blk
    sim_ref[...] = sim_blk
    bkn_ref[...] = jnp.broadcast_to(krow_ref[...], (BLK, K * D))
    pe_ref[...] = jnp.concatenate(
        [jnp.broadcast_to(prow_ref[...], (BLK, K * LP * D)), x_blk], axis=1)


def kernel(x, prompt, prompt_key):
    pf = prompt.reshape(P, LP * D)
    outs = pl.pallas_call(
        _body,
        grid=(GRID,),
        in_specs=[
            pl.BlockSpec((B, D), lambda i: (0, 0)),
            pl.BlockSpec((P, LP * D), lambda i: (0, 0)),
            pl.BlockSpec((P, D), lambda i: (0, 0)),
        ],
        out_specs=[
            pl.BlockSpec((BLK, K), lambda i: (i, 0)),
            pl.BlockSpec((P, D), lambda i: (0, 0)),
            pl.BlockSpec((BLK, D), lambda i: (i, 0)),
            pl.BlockSpec((BLK, P), lambda i: (i, 0)),
            pl.BlockSpec((BLK, K * D), lambda i: (i, 0)),
            pl.BlockSpec((1, 1), lambda i: (0, 0)),
            pl.BlockSpec((BLK, PE_W), lambda i: (i, 0)),
        ],
        out_shape=[
            jax.ShapeDtypeStruct((B, K), jnp.int32),
            jax.ShapeDtypeStruct((P, D), jnp.float32),
            jax.ShapeDtypeStruct((B, D), jnp.float32),
            jax.ShapeDtypeStruct((B, P), jnp.float32),
            jax.ShapeDtypeStruct((B, K * D), jnp.float32),
            jax.ShapeDtypeStruct((1, 1), jnp.float32),
            jax.ShapeDtypeStruct((B, PE_W), jnp.float32),
        ],
        scratch_shapes=[
            pltpu.VMEM((1, K * LP * D), jnp.float32),
            pltpu.VMEM((1, K * D), jnp.float32),
            pltpu.VMEM((1, K), jnp.int32),
            pltpu.VMEM((P, D), jnp.float32),
        ],
    )(x, pf, prompt_key)
    idx_b, pn, xn, sim, bkn, rs, pe = outs
    return (idx_b, pn, xn, sim, bkn.reshape(B, K, D), rs[0, 0], pe)
